# baseline (speedup 1.0000x reference)
import jax
import jax.numpy as jnp
from jax import lax
from jax.experimental import pallas as pl
from jax.experimental.pallas import tpu as pltpu
from jax.experimental.pallas import tpu_sc as plsc

_NC = 2
_NS = 16
_NW = _NC * _NS

_B, _L = 4096, 200
_N = _B * _L
_V = 21
_D = 21
_DP = 24
_IW = 128
_PER_W = _N // _NW
_TILES_W = _PER_W // _IW
_G = 20
_NGRP = _TILES_W // _G
_R = 2048
_BPT = _IW * _DP // 1024


def _body(idx_hbm, table_hbm, out_hbm, idx_all, sem_g, sem_o, *row_bufs):
    wid = lax.axis_index("s") * _NC + lax.axis_index("c")
    pltpu.sync_copy(idx_hbm.at[pl.ds(wid * (_TILES_W // 8), _TILES_W // 8)],
                    idx_all)

    def step(i, carry):
        t0 = i * _G
        gathers = [
            pltpu.async_copy(
                table_hbm.at[idx_all.at[(t0 + j) // 8, (t0 + j) % 8]],
                row_bufs[j], sem_g)
            for j in range(_G)
        ]
        for g in gathers:
            g.wait()
        base = wid * _PER_W + t0 * _IW
        writes = [
            pltpu.async_copy(row_bufs[j],
                             out_hbm.at[pl.ds(base + j * _IW, _IW)], sem_o)
            for j in range(_G)
        ]
        for w in writes:
            w.wait()
        return carry

    lax.fori_loop(0, _NGRP, step, 0)


def kernel(sequence_indices, table):
    rep_off = _V * (jnp.arange(_N, dtype=jnp.int32) % _R)
    idx_rows = (sequence_indices.reshape(_N) + rep_off).reshape(
        _N // 1024, 8, _IW)
    table_padded = jnp.tile(jnp.pad(table, ((0, 0), (0, _DP - _D))),
                            (_R, 1))
    mesh = plsc.VectorSubcoreMesh(
        core_axis_name="c", subcore_axis_name="s",
        num_cores=_NC, num_subcores=_NS,
    )
    k = pl.kernel(
        _body,
        out_type=jax.ShapeDtypeStruct((_N, _DP), jnp.float32),
        mesh=mesh,
        scratch_types=[
            pltpu.VMEM((_TILES_W // 8, 8, _IW), jnp.int32),
            pltpu.SemaphoreType.DMA,
            pltpu.SemaphoreType.DMA,
        ] + [pltpu.VMEM((_IW, _DP), jnp.float32) for _ in range(_G)],
        compiler_params=pltpu.CompilerParams(use_tc_tiling_on_sc=False),
    )
    out = k(idx_rows, table_padded)
    return out[:, :_D].reshape(_B, _L, _D)


# speedup vs baseline: 1.0430x; 1.0430x over previous
import jax
import jax.numpy as jnp
from jax import lax
from jax.experimental import pallas as pl
from jax.experimental.pallas import tpu as pltpu
from jax.experimental.pallas import tpu_sc as plsc

_NC = 2
_NS = 16
_NW = _NC * _NS

_B, _L = 4096, 200
_N = _B * _L
_V = 21
_D = 21
_DP = 24
_IW = 128
_PER_W = _N // _NW
_TILES_W = _PER_W // _IW
_G = 20
_NGRP = _TILES_W // _G
_R = 256
_BPT = _IW * _DP // 1024


def _body(idx_hbm, table_hbm, out_hbm, idx_all, sem_g, sem_o, *row_bufs):
    wid = lax.axis_index("s") * _NC + lax.axis_index("c")
    pltpu.sync_copy(idx_hbm.at[pl.ds(wid * (_TILES_W // 8), _TILES_W // 8)],
                    idx_all)

    def step(i, carry):
        t0 = i * _G
        gathers = [
            pltpu.async_copy(
                table_hbm.at[idx_all.at[(t0 + j) // 8, (t0 + j) % 8]],
                row_bufs[j], sem_g)
            for j in range(_G)
        ]
        for g in gathers:
            g.wait()
        base = wid * _PER_W + t0 * _IW
        writes = [
            pltpu.async_copy(row_bufs[j],
                             out_hbm.at[pl.ds(base + j * _IW, _IW)], sem_o)
            for j in range(_G)
        ]
        for w in writes:
            w.wait()
        return carry

    lax.fori_loop(0, _NGRP, step, 0)


def kernel(sequence_indices, table):
    rep_off = _V * (jnp.arange(_N, dtype=jnp.int32) % _R)
    idx_rows = (sequence_indices.reshape(_N) + rep_off).reshape(
        _N // 1024, 8, _IW)
    table_padded = jnp.tile(jnp.pad(table, ((0, 0), (0, _DP - _D))),
                            (_R, 1))
    mesh = plsc.VectorSubcoreMesh(
        core_axis_name="c", subcore_axis_name="s",
        num_cores=_NC, num_subcores=_NS,
    )
    k = pl.kernel(
        _body,
        out_type=jax.ShapeDtypeStruct((_N, _DP), jnp.float32),
        mesh=mesh,
        scratch_types=[
            pltpu.VMEM((_TILES_W // 8, 8, _IW), jnp.int32),
            pltpu.SemaphoreType.DMA,
            pltpu.SemaphoreType.DMA,
        ] + [pltpu.VMEM((_IW, _DP), jnp.float32) for _ in range(_G)],
        compiler_params=pltpu.CompilerParams(use_tc_tiling_on_sc=False),
    )
    out = k(idx_rows, table_padded)
    return out[:, :_D].reshape(_B, _L, _D)


# speedup vs baseline: 1.0434x; 1.0004x over previous
import jax
import jax.numpy as jnp
from jax import lax
from jax.experimental import pallas as pl
from jax.experimental.pallas import tpu as pltpu
from jax.experimental.pallas import tpu_sc as plsc

_NC = 2
_NS = 16
_NW = _NC * _NS

_B, _L = 4096, 200
_N = _B * _L
_V = 21
_D = 21
_DP = 24
_IW = 128
_PER_W = _N // _NW
_TILES_W = _PER_W // _IW
_G = 20
_NGRP = _TILES_W // _G
_R = 256
_BPT = _IW * _DP // 1024


def _body(idx_hbm, table_hbm, out_hbm, idx_all, sem_g, sem_o, *row_bufs):
    wid = lax.axis_index("s") * _NC + lax.axis_index("c")
    pltpu.sync_copy(idx_hbm.at[pl.ds(wid * (_TILES_W // 8), _TILES_W // 8)],
                    idx_all)

    def drain_writes():
        for j in range(_G):
            pltpu.make_async_copy(row_bufs[j], out_hbm.at[pl.ds(0, _IW)],
                                  sem_o).wait()

    def step(i, carry):
        @pl.when(i > 0)
        def _():
            drain_writes()

        t0 = i * _G
        gathers = [
            pltpu.async_copy(
                table_hbm.at[idx_all.at[(t0 + j) // 8, (t0 + j) % 8]],
                row_bufs[j], sem_g)
            for j in range(_G)
        ]
        for g in gathers:
            g.wait()
        base = wid * _PER_W + t0 * _IW
        for j in range(_G):
            pltpu.async_copy(row_bufs[j],
                             out_hbm.at[pl.ds(base + j * _IW, _IW)], sem_o)
        return carry

    lax.fori_loop(0, _NGRP, step, 0)
    drain_writes()


def kernel(sequence_indices, table):
    rep_off = _V * (jnp.arange(_N, dtype=jnp.int32) % _R)
    idx_rows = (sequence_indices.reshape(_N) + rep_off).reshape(
        _N // 1024, 8, _IW)
    table_padded = jnp.tile(jnp.pad(table, ((0, 0), (0, _DP - _D))),
                            (_R, 1))
    mesh = plsc.VectorSubcoreMesh(
        core_axis_name="c", subcore_axis_name="s",
        num_cores=_NC, num_subcores=_NS,
    )
    k = pl.kernel(
        _body,
        out_type=jax.ShapeDtypeStruct((_N, _DP), jnp.float32),
        mesh=mesh,
        scratch_types=[
            pltpu.VMEM((_TILES_W // 8, 8, _IW), jnp.int32),
            pltpu.SemaphoreType.DMA,
            pltpu.SemaphoreType.DMA,
        ] + [pltpu.VMEM((_IW, _DP), jnp.float32) for _ in range(_G)],
        compiler_params=pltpu.CompilerParams(use_tc_tiling_on_sc=False),
    )
    out = k(idx_rows, table_padded)
    return out[:, :_D].reshape(_B, _L, _D)
